# baseline (speedup 1.0000x reference)
import functools

import jax
import jax.numpy as jnp
from jax import lax
from jax.experimental import pallas as pl
from jax.experimental.pallas import tpu as pltpu
from jax.experimental.pallas import tpu_sc as plsc

_N = 10000
_E = 320000
_D = 128
_B = 64
_BN = 2000
_NB = _N // _BN
_NW = 32
_EPW = _E // _NW
_L = 16


def _proj_body(x_ref, w_ref, y_ref):
    y_ref[...] = lax.dot_general(
        w_ref[...], x_ref[...], (((1,), (1,)), ((), ())),
        precision=lax.Precision.HIGHEST,
        preferred_element_type=jnp.float32).reshape(1, 1, _BN)


def _proj(x, w_row):
    return pl.pallas_call(
        _proj_body,
        grid=(_NB,),
        in_specs=[pl.BlockSpec((_BN, _D), lambda i: (i, 0)),
                  pl.BlockSpec((1, _D), lambda i: (0, 0))],
        out_specs=pl.BlockSpec((1, 1, _BN), lambda i: (i, 0, 0)),
        out_shape=jax.ShapeDtypeStruct((_NB, 1, _BN), jnp.float32),
    )(x, w_row)


def _edge_body(y_hbm, src_hbm, dst_hbm, out_hbm, ytab, srcv, dstv, acc,
               sem_y, sem_s, sem_d):
    wid = lax.axis_index("s") * 2 + lax.axis_index("c")
    base = wid * _EPW
    cp_y = pltpu.async_copy(y_hbm, ytab, sem_y)
    cp_s = pltpu.async_copy(src_hbm.at[pl.ds(base, _EPW)], srcv, sem_s)
    cp_d = pltpu.async_copy(dst_hbm.at[pl.ds(base, _EPW)], dstv, sem_d)

    zero = jnp.zeros((_L,), jnp.float32)

    def zbody(i, c):
        acc[pl.ds(i * _L, _L)] = zero
        return c

    lax.fori_loop(0, _N // _L, zbody, 0, unroll=8)
    cp_y.wait()
    cp_s.wait()
    cp_d.wait()

    def ebody(i, c):
        s = srcv[pl.ds(i * _L, _L)]
        d = dstv[pl.ds(i * _L, _L)]
        v = plsc.load_gather(ytab, [s])
        plsc.addupdate_scatter(acc, [d], v)
        return c

    lax.fori_loop(0, _EPW // _L, ebody, 0, unroll=16)
    pltpu.sync_copy(acc, out_hbm.at[wid])


def _edge(y_flat, src, dst):
    mesh = plsc.VectorSubcoreMesh(core_axis_name="c", subcore_axis_name="s")
    f = pl.kernel(
        _edge_body,
        mesh=mesh,
        compiler_params=pltpu.CompilerParams(needs_layout_passes=False,
                                             use_tc_tiling_on_sc=False),
        out_type=jax.ShapeDtypeStruct((_NW, _N), jnp.float32),
        scratch_types=[pltpu.VMEM((_N,), jnp.float32),
                       pltpu.VMEM((_EPW,), jnp.int32),
                       pltpu.VMEM((_EPW,), jnp.int32),
                       pltpu.VMEM((_N,), jnp.float32),
                       pltpu.SemaphoreType.DMA,
                       pltpu.SemaphoreType.DMA,
                       pltpu.SemaphoreType.DMA],
    )
    return f(y_flat, src, dst)


def _pool_body(x_ref, parts_ref, batch_ref, wroot_ref, brel_ref, out_ref,
               m_ref, d_ref, g_ref):
    i = pl.program_id(0)

    @pl.when(i == 0)
    def _init():
        m_ref[...] = jnp.full((_B, 1), -jnp.inf, jnp.float32)
        d_ref[...] = jnp.zeros((_B, 1), jnp.float32)
        g_ref[...] = jnp.zeros((_B, _D), jnp.float32)

    x = x_ref[...]
    parts = parts_ref[...].reshape(_NW, _BN)
    e_row = jnp.sum(parts, axis=0, keepdims=True)
    yroot_row = lax.dot_general(
        wroot_ref[...], x, (((1,), (1,)), ((), ())),
        precision=lax.Precision.HIGHEST,
        preferred_element_type=jnp.float32)
    xc = e_row + yroot_row + brel_ref[...]

    b_row = batch_ref[...].reshape(1, _BN)
    gids = lax.broadcasted_iota(jnp.int32, (_B, _BN), 0)
    P = b_row == gids
    Pf = P.astype(jnp.float32)

    m_old = m_ref[...]
    blk_max = jnp.max(jnp.where(P, xc, -jnp.inf), axis=1, keepdims=True)
    m_new = jnp.maximum(m_old, blk_max)
    scale = jnp.exp(jnp.where(m_new == -jnp.inf, 0.0, m_old - m_new))
    m_safe = jnp.where(m_new == -jnp.inf, 0.0, m_new)
    mrow = lax.dot_general(
        m_safe, Pf, (((0,), (0,)), ((), ())),
        precision=lax.Precision.HIGHEST,
        preferred_element_type=jnp.float32)
    ex_row = jnp.exp(xc - mrow)
    EX = Pf * ex_row
    d_ref[...] = d_ref[...] * scale + jnp.sum(EX, axis=1, keepdims=True)
    g_ref[...] = g_ref[...] * scale + jnp.dot(
        EX, x, precision=lax.Precision.HIGHEST,
        preferred_element_type=jnp.float32)
    m_ref[...] = m_new

    @pl.when(i == _NB - 1)
    def _fin():
        out_ref[...] = g_ref[...] / (d_ref[...] + 1e-16)


def _pool(x, parts, batch3, wroot_row, brel):
    return pl.pallas_call(
        _pool_body,
        grid=(_NB,),
        in_specs=[pl.BlockSpec((_BN, _D), lambda i: (i, 0)),
                  pl.BlockSpec((_NW, 1, 1, _BN), lambda i: (0, i, 0, 0)),
                  pl.BlockSpec((1, 1, _BN), lambda i: (i, 0, 0)),
                  pl.BlockSpec((1, _D), lambda i: (0, 0)),
                  pl.BlockSpec((1, 1), lambda i: (0, 0))],
        out_specs=pl.BlockSpec((_B, _D), lambda i: (0, 0)),
        out_shape=jax.ShapeDtypeStruct((_B, _D), jnp.float32),
        scratch_shapes=[pltpu.VMEM((_B, 1), jnp.float32),
                        pltpu.VMEM((_B, 1), jnp.float32),
                        pltpu.VMEM((_B, _D), jnp.float32)],
    )(x, parts, batch3, wroot_row, brel)


def kernel(x, edge_index, batch, W_rel, b_rel, W_root):
    y_rel = _proj(x, W_rel.reshape(1, _D))
    parts = _edge(y_rel.reshape(_N), edge_index[0], edge_index[1])
    parts = parts.reshape(_NW, _NB, 1, _BN)
    batch3 = batch.reshape(_NB, 1, _BN)
    gx = _pool(x, parts, batch3, W_root.reshape(1, _D),
               b_rel.reshape(1, 1))
    return gx


# speedup vs baseline: 1.1732x; 1.1732x over previous
import functools

import jax
import jax.numpy as jnp
from jax import lax
from jax.experimental import pallas as pl
from jax.experimental.pallas import tpu as pltpu
from jax.experimental.pallas import tpu_sc as plsc

_N = 10000
_E = 320000
_D = 128
_B = 64
_BN = 2000
_NB = _N // _BN
_NW = 32
_EPW = _E // _NW
_L = 16


def _proj_body(x_ref, w_ref, y_ref):
    y_ref[...] = lax.dot_general(
        w_ref[...], x_ref[...], (((1,), (1,)), ((), ())),
        precision=lax.Precision.HIGHEST,
        preferred_element_type=jnp.float32).reshape(1, 1, _BN)


def _proj(x, w_row):
    return pl.pallas_call(
        _proj_body,
        grid=(_NB,),
        in_specs=[pl.BlockSpec((_BN, _D), lambda i: (i, 0)),
                  pl.BlockSpec((1, _D), lambda i: (0, 0))],
        out_specs=pl.BlockSpec((1, 1, _BN), lambda i: (i, 0, 0)),
        out_shape=jax.ShapeDtypeStruct((_NB, 1, _BN), jnp.float32),
    )(x, w_row)


def _edge_body(y_hbm, ei_hbm, out_hbm, ytab, srcv, dstv, acc,
               sem_y, sem_s, sem_d):
    wid = lax.axis_index("s") * 2 + lax.axis_index("c")
    base = wid * _EPW
    cp_y = pltpu.async_copy(y_hbm, ytab, sem_y)
    cp_s = pltpu.async_copy(ei_hbm.at[0, pl.ds(base, _EPW)], srcv, sem_s)
    cp_d = pltpu.async_copy(ei_hbm.at[1, pl.ds(base, _EPW)], dstv, sem_d)

    zero = jnp.zeros((_L,), jnp.float32)

    def zbody(i, c):
        acc[pl.ds(i * _L, _L)] = zero
        return c

    lax.fori_loop(0, _N // _L, zbody, 0, unroll=8)
    cp_y.wait()
    cp_s.wait()
    cp_d.wait()

    def ebody(i, c):
        s = srcv[pl.ds(i * _L, _L)]
        d = dstv[pl.ds(i * _L, _L)]
        v = plsc.load_gather(ytab, [s])
        plsc.addupdate_scatter(acc, [d], v)
        return c

    lax.fori_loop(0, _EPW // _L, ebody, 0, unroll=16)
    pltpu.sync_copy(acc, out_hbm.at[wid])


def _edge(y_flat, edge_index):
    mesh = plsc.VectorSubcoreMesh(core_axis_name="c", subcore_axis_name="s")
    f = pl.kernel(
        _edge_body,
        mesh=mesh,
        compiler_params=pltpu.CompilerParams(needs_layout_passes=False,
                                             use_tc_tiling_on_sc=False),
        out_type=jax.ShapeDtypeStruct((_NW, _N), jnp.float32),
        scratch_types=[pltpu.VMEM((_N,), jnp.float32),
                       pltpu.VMEM((_EPW,), jnp.int32),
                       pltpu.VMEM((_EPW,), jnp.int32),
                       pltpu.VMEM((_N,), jnp.float32),
                       pltpu.SemaphoreType.DMA,
                       pltpu.SemaphoreType.DMA,
                       pltpu.SemaphoreType.DMA],
    )
    return f(y_flat, edge_index)


def _pool_body(x_ref, parts_ref, batch_ref, wroot_ref, brel_ref, out_ref,
               m_ref, d_ref, g_ref):
    i = pl.program_id(0)

    @pl.when(i == 0)
    def _init():
        m_ref[...] = jnp.full((_B, 1), -jnp.inf, jnp.float32)
        d_ref[...] = jnp.zeros((_B, 1), jnp.float32)
        g_ref[...] = jnp.zeros((_B, _D), jnp.float32)

    x = x_ref[...]
    parts = parts_ref[...].reshape(_NW, _BN)
    e_row = jnp.sum(parts, axis=0, keepdims=True)
    yroot_row = lax.dot_general(
        wroot_ref[...], x, (((1,), (1,)), ((), ())),
        precision=lax.Precision.HIGHEST,
        preferred_element_type=jnp.float32)
    xc = e_row + yroot_row + brel_ref[...]

    b_row = batch_ref[...].reshape(1, _BN)
    gids = lax.broadcasted_iota(jnp.int32, (_B, _BN), 0)
    P = b_row == gids
    Pf = P.astype(jnp.float32)

    m_old = m_ref[...]
    blk_max = jnp.max(jnp.where(P, xc, -jnp.inf), axis=1, keepdims=True)
    m_new = jnp.maximum(m_old, blk_max)
    scale = jnp.exp(jnp.where(m_new == -jnp.inf, 0.0, m_old - m_new))
    m_safe = jnp.where(m_new == -jnp.inf, 0.0, m_new)
    mrow = lax.dot_general(
        m_safe, Pf, (((0,), (0,)), ((), ())),
        precision=lax.Precision.HIGHEST,
        preferred_element_type=jnp.float32)
    ex_row = jnp.exp(xc - mrow)
    EX = Pf * ex_row
    d_ref[...] = d_ref[...] * scale + jnp.sum(EX, axis=1, keepdims=True)
    g_ref[...] = g_ref[...] * scale + jnp.dot(
        EX, x, precision=lax.Precision.HIGHEST,
        preferred_element_type=jnp.float32)
    m_ref[...] = m_new

    @pl.when(i == _NB - 1)
    def _fin():
        out_ref[...] = g_ref[...] / (d_ref[...] + 1e-16)


def _pool(x, parts, batch3, wroot_row, brel):
    return pl.pallas_call(
        _pool_body,
        grid=(_NB,),
        in_specs=[pl.BlockSpec((_BN, _D), lambda i: (i, 0)),
                  pl.BlockSpec((_NW, 1, 1, _BN), lambda i: (0, i, 0, 0)),
                  pl.BlockSpec((1, 1, _BN), lambda i: (i, 0, 0)),
                  pl.BlockSpec((1, _D), lambda i: (0, 0)),
                  pl.BlockSpec((1, 1), lambda i: (0, 0))],
        out_specs=pl.BlockSpec((_B, _D), lambda i: (0, 0)),
        out_shape=jax.ShapeDtypeStruct((_B, _D), jnp.float32),
        scratch_shapes=[pltpu.VMEM((_B, 1), jnp.float32),
                        pltpu.VMEM((_B, 1), jnp.float32),
                        pltpu.VMEM((_B, _D), jnp.float32)],
    )(x, parts, batch3, wroot_row, brel)


def kernel(x, edge_index, batch, W_rel, b_rel, W_root):
    y_rel = _proj(x, W_rel.reshape(1, _D))
    parts = _edge(y_rel.reshape(_N), edge_index)
    parts = parts.reshape(_NW, _NB, 1, _BN)
    batch3 = batch.reshape(_NB, 1, _BN)
    gx = _pool(x, parts, batch3, W_root.reshape(1, _D),
               b_rel.reshape(1, 1))
    return gx


# speedup vs baseline: 1.2696x; 1.0822x over previous
import functools

import jax
import jax.numpy as jnp
from jax import lax
from jax.experimental import pallas as pl
from jax.experimental.pallas import tpu as pltpu
from jax.experimental.pallas import tpu_sc as plsc

_N = 10000
_E = 320000
_D = 128
_B = 64
_BN = 2000
_NB = _N // _BN
_NW = 32
_EPW = _E // _NW
_L = 16


def _proj_body(x_ref, w_ref, y_ref):
    y_ref[...] = lax.dot_general(
        w_ref[...], x_ref[...], (((1,), (1,)), ((), ())),
        precision=lax.Precision.HIGHEST,
        preferred_element_type=jnp.float32).reshape(1, 2, _BN)


def _proj(x, w_rows):
    return pl.pallas_call(
        _proj_body,
        grid=(_NB,),
        in_specs=[pl.BlockSpec((_BN, _D), lambda i: (i, 0)),
                  pl.BlockSpec((2, _D), lambda i: (0, 0))],
        out_specs=pl.BlockSpec((1, 2, _BN), lambda i: (i, 0, 0)),
        out_shape=jax.ShapeDtypeStruct((_NB, 2, _BN), jnp.float32),
    )(x, w_rows)


def _edge_body(y_hbm, ei_hbm, out_hbm, ytab, srcv, dstv, acc,
               sem_y, sem_s, sem_d):
    wid = lax.axis_index("s") * 2 + lax.axis_index("c")
    base = wid * _EPW
    cp_y = [pltpu.async_copy(y_hbm.at[j, 0], ytab.at[pl.ds(j * _BN, _BN)],
                             sem_y) for j in range(_NB)]
    cp_s = pltpu.async_copy(ei_hbm.at[0, pl.ds(base, _EPW)], srcv, sem_s)
    cp_d = pltpu.async_copy(ei_hbm.at[1, pl.ds(base, _EPW)], dstv, sem_d)

    zero = jnp.zeros((_L,), jnp.float32)

    def zbody(i, c):
        acc[pl.ds(i * _L, _L)] = zero
        return c

    lax.fori_loop(0, _N // _L, zbody, 0, unroll=8)
    for cp in cp_y:
        cp.wait()
    cp_s.wait()
    cp_d.wait()

    def ebody(i, c):
        s = srcv[pl.ds(i * _L, _L)]
        d = dstv[pl.ds(i * _L, _L)]
        v = plsc.load_gather(ytab, [s])
        plsc.addupdate_scatter(acc, [d], v)
        return c

    lax.fori_loop(0, _EPW // _L, ebody, 0, unroll=16)
    pltpu.sync_copy(acc, out_hbm.at[wid])


def _edge(y_flat, edge_index):
    mesh = plsc.VectorSubcoreMesh(core_axis_name="c", subcore_axis_name="s")
    f = pl.kernel(
        _edge_body,
        mesh=mesh,
        compiler_params=pltpu.CompilerParams(needs_layout_passes=False,
                                             use_tc_tiling_on_sc=False),
        out_type=jax.ShapeDtypeStruct((_NW, _N), jnp.float32),
        scratch_types=[pltpu.VMEM((_N,), jnp.float32),
                       pltpu.VMEM((_EPW,), jnp.int32),
                       pltpu.VMEM((_EPW,), jnp.int32),
                       pltpu.VMEM((_N,), jnp.float32),
                       pltpu.SemaphoreType.DMA,
                       pltpu.SemaphoreType.DMA,
                       pltpu.SemaphoreType.DMA],
    )
    return f(y_flat, edge_index)


def _pool_body(x_ref, parts_ref, batch_ref, y2_ref, brel_ref, out_ref,
               m_ref, d_ref, g_ref):
    i = pl.program_id(0)

    @pl.when(i == 0)
    def _init():
        m_ref[...] = jnp.full((_B, 1), -jnp.inf, jnp.float32)
        d_ref[...] = jnp.zeros((_B, 1), jnp.float32)
        g_ref[...] = jnp.zeros((_B, _D), jnp.float32)

    x = x_ref[...]
    parts = parts_ref[...].reshape(_NW, _BN)
    e_row = jnp.sum(parts, axis=0, keepdims=True)
    yroot_row = y2_ref[0, 1, :].reshape(1, _BN)
    xc = e_row + yroot_row + brel_ref[...]

    b_row = batch_ref[...].reshape(1, _BN)
    gids = lax.broadcasted_iota(jnp.int32, (_B, _BN), 0)
    P = b_row == gids
    Pf = P.astype(jnp.float32)

    m_old = m_ref[...]
    blk_max = jnp.max(jnp.where(P, xc, -jnp.inf), axis=1, keepdims=True)
    m_new = jnp.maximum(m_old, blk_max)
    scale = jnp.exp(jnp.where(m_new == -jnp.inf, 0.0, m_old - m_new))
    m_safe = jnp.where(m_new == -jnp.inf, 0.0, m_new)
    mrow = lax.dot_general(
        m_safe, Pf, (((0,), (0,)), ((), ())),
        precision=lax.Precision.HIGHEST,
        preferred_element_type=jnp.float32)
    ex_row = jnp.exp(xc - mrow)
    EX = Pf * ex_row
    d_ref[...] = d_ref[...] * scale + jnp.sum(EX, axis=1, keepdims=True)
    g_ref[...] = g_ref[...] * scale + jnp.dot(
        EX, x, precision=lax.Precision.HIGHEST,
        preferred_element_type=jnp.float32)
    m_ref[...] = m_new

    @pl.when(i == _NB - 1)
    def _fin():
        out_ref[...] = g_ref[...] / (d_ref[...] + 1e-16)


def _pool(x, parts, batch3, y2, brel):
    return pl.pallas_call(
        _pool_body,
        grid=(_NB,),
        in_specs=[pl.BlockSpec((_BN, _D), lambda i: (i, 0)),
                  pl.BlockSpec((_NW, 1, 1, _BN), lambda i: (0, i, 0, 0)),
                  pl.BlockSpec((1, 1, _BN), lambda i: (i, 0, 0)),
                  pl.BlockSpec((1, 2, _BN), lambda i: (i, 0, 0)),
                  pl.BlockSpec((1, 1), lambda i: (0, 0))],
        out_specs=pl.BlockSpec((_B, _D), lambda i: (0, 0)),
        out_shape=jax.ShapeDtypeStruct((_B, _D), jnp.float32),
        scratch_shapes=[pltpu.VMEM((_B, 1), jnp.float32),
                        pltpu.VMEM((_B, 1), jnp.float32),
                        pltpu.VMEM((_B, _D), jnp.float32)],
    )(x, parts, batch3, y2, brel)


def kernel(x, edge_index, batch, W_rel, b_rel, W_root):
    w_rows = jnp.concatenate([W_rel.reshape(1, _D), W_root.reshape(1, _D)],
                             axis=0)
    y2 = _proj(x, w_rows)
    parts = _edge(y2, edge_index)
    parts = parts.reshape(_NW, _NB, 1, _BN)
    batch3 = batch.reshape(_NB, 1, _BN)
    gx = _pool(x, parts, batch3, y2, b_rel.reshape(1, 1))
    return gx


# speedup vs baseline: 1.3703x; 1.0794x over previous
import functools

import jax
import jax.numpy as jnp
from jax import lax
from jax.experimental import pallas as pl
from jax.experimental.pallas import tpu as pltpu
from jax.experimental.pallas import tpu_sc as plsc

_N = 10000
_E = 320000
_D = 128
_B = 64
_BN = 2000
_NB = _N // _BN
_NW = 32
_EPW = _E // _NW
_L = 16


def _proj_body(x_ref, w_ref, y_ref):
    y_ref[...] = lax.dot_general(
        w_ref[...], x_ref[...], (((1,), (1,)), ((), ())),
        precision=lax.Precision.HIGHEST,
        preferred_element_type=jnp.float32).reshape(1, 2, _BN)


def _proj(x, w_rows):
    return pl.pallas_call(
        _proj_body,
        grid=(_NB,),
        in_specs=[pl.BlockSpec((_BN, _D), lambda i: (i, 0)),
                  pl.BlockSpec((2, _D), lambda i: (0, 0))],
        out_specs=pl.BlockSpec((1, 2, _BN), lambda i: (i, 0, 0)),
        out_shape=jax.ShapeDtypeStruct((_NB, 2, _BN), jnp.float32),
    )(x, w_rows)


def _edge_body(y_hbm, ei_hbm, out_hbm, ytab, srcv, dstv, acc,
               sem_y, sem_s, sem_d):
    wid = lax.axis_index("s") * 2 + lax.axis_index("c")
    base = wid * _EPW
    cp_y = [pltpu.async_copy(y_hbm.at[j, 0], ytab.at[pl.ds(j * _BN, _BN)],
                             sem_y) for j in range(_NB)]
    cp_s = pltpu.async_copy(ei_hbm.at[0, pl.ds(base, _EPW)], srcv, sem_s)
    cp_d = pltpu.async_copy(ei_hbm.at[1, pl.ds(base, _EPW)], dstv, sem_d)

    zero = jnp.zeros((_L,), jnp.float32)

    @plsc.parallel_loop(0, _N // _L, unroll=8)
    def _zero(i):
        acc[pl.ds(i * _L, _L)] = zero

    for cp in cp_y:
        cp.wait()
    cp_s.wait()
    cp_d.wait()

    @plsc.parallel_loop(0, _EPW // _L, unroll=16)
    def _edge_iter(i):
        s = srcv[pl.ds(i * _L, _L)]
        d = dstv[pl.ds(i * _L, _L)]
        v = plsc.load_gather(ytab, [s])
        plsc.addupdate_scatter(acc, [d], v)
    pltpu.sync_copy(acc, out_hbm.at[wid])


def _edge(y_flat, edge_index):
    mesh = plsc.VectorSubcoreMesh(core_axis_name="c", subcore_axis_name="s")
    f = pl.kernel(
        _edge_body,
        mesh=mesh,
        compiler_params=pltpu.CompilerParams(needs_layout_passes=False,
                                             use_tc_tiling_on_sc=False),
        out_type=jax.ShapeDtypeStruct((_NW, _N), jnp.float32),
        scratch_types=[pltpu.VMEM((_N,), jnp.float32),
                       pltpu.VMEM((_EPW,), jnp.int32),
                       pltpu.VMEM((_EPW,), jnp.int32),
                       pltpu.VMEM((_N,), jnp.float32),
                       pltpu.SemaphoreType.DMA,
                       pltpu.SemaphoreType.DMA,
                       pltpu.SemaphoreType.DMA],
    )
    return f(y_flat, edge_index)


def _pool_body(x_ref, parts_ref, batch_ref, y2_ref, brel_ref, out_ref,
               m_ref, d_ref, g_ref):
    i = pl.program_id(0)

    @pl.when(i == 0)
    def _init():
        m_ref[...] = jnp.full((_B, 1), -jnp.inf, jnp.float32)
        d_ref[...] = jnp.zeros((_B, 1), jnp.float32)
        g_ref[...] = jnp.zeros((_B, _D), jnp.float32)

    x = x_ref[...]
    parts = parts_ref[...].reshape(_NW, _BN)
    e_row = jnp.sum(parts, axis=0, keepdims=True)
    yroot_row = y2_ref[0, 1, :].reshape(1, _BN)
    xc = e_row + yroot_row + brel_ref[...]

    b_row = batch_ref[...].reshape(1, _BN)
    gids = lax.broadcasted_iota(jnp.int32, (_B, _BN), 0)
    P = b_row == gids
    Pf = P.astype(jnp.float32)

    m_old = m_ref[...]
    blk_max = jnp.max(jnp.where(P, xc, -jnp.inf), axis=1, keepdims=True)
    m_new = jnp.maximum(m_old, blk_max)
    scale = jnp.exp(jnp.where(m_new == -jnp.inf, 0.0, m_old - m_new))
    m_safe = jnp.where(m_new == -jnp.inf, 0.0, m_new)
    mrow = lax.dot_general(
        m_safe, Pf, (((0,), (0,)), ((), ())),
        precision=lax.Precision.HIGHEST,
        preferred_element_type=jnp.float32)
    ex_row = jnp.exp(xc - mrow)
    EX = Pf * ex_row
    d_ref[...] = d_ref[...] * scale + jnp.sum(EX, axis=1, keepdims=True)
    g_ref[...] = g_ref[...] * scale + jnp.dot(
        EX, x, precision=lax.Precision.HIGHEST,
        preferred_element_type=jnp.float32)
    m_ref[...] = m_new

    @pl.when(i == _NB - 1)
    def _fin():
        out_ref[...] = g_ref[...] / (d_ref[...] + 1e-16)


def _pool(x, parts, batch3, y2, brel):
    return pl.pallas_call(
        _pool_body,
        grid=(_NB,),
        in_specs=[pl.BlockSpec((_BN, _D), lambda i: (i, 0)),
                  pl.BlockSpec((_NW, 1, 1, _BN), lambda i: (0, i, 0, 0)),
                  pl.BlockSpec((1, 1, _BN), lambda i: (i, 0, 0)),
                  pl.BlockSpec((1, 2, _BN), lambda i: (i, 0, 0)),
                  pl.BlockSpec((1, 1), lambda i: (0, 0))],
        out_specs=pl.BlockSpec((_B, _D), lambda i: (0, 0)),
        out_shape=jax.ShapeDtypeStruct((_B, _D), jnp.float32),
        scratch_shapes=[pltpu.VMEM((_B, 1), jnp.float32),
                        pltpu.VMEM((_B, 1), jnp.float32),
                        pltpu.VMEM((_B, _D), jnp.float32)],
    )(x, parts, batch3, y2, brel)


def kernel(x, edge_index, batch, W_rel, b_rel, W_root):
    w_rows = jnp.concatenate([W_rel.reshape(1, _D), W_root.reshape(1, _D)],
                             axis=0)
    y2 = _proj(x, w_rows)
    parts = _edge(y2, edge_index)
    parts = parts.reshape(_NW, _NB, 1, _BN)
    batch3 = batch.reshape(_NB, 1, _BN)
    gx = _pool(x, parts, batch3, y2, b_rel.reshape(1, 1))
    return gx


# speedup vs baseline: 1.3989x; 1.0208x over previous
import functools

import jax
import jax.numpy as jnp
from jax import lax
from jax.experimental import pallas as pl
from jax.experimental.pallas import tpu as pltpu
from jax.experimental.pallas import tpu_sc as plsc

_N = 10000
_E = 320000
_D = 128
_B = 64
_BN = 5000
_NB = _N // _BN
_NW = 32
_EPW = _E // _NW
_L = 16


def _proj_body(x_ref, w_ref, y_ref):
    y_ref[...] = lax.dot_general(
        w_ref[...], x_ref[...], (((1,), (1,)), ((), ())),
        precision=lax.Precision.HIGHEST,
        preferred_element_type=jnp.float32).reshape(1, 2, _BN)


def _proj(x, w_rows):
    return pl.pallas_call(
        _proj_body,
        grid=(_NB,),
        in_specs=[pl.BlockSpec((_BN, _D), lambda i: (i, 0)),
                  pl.BlockSpec((2, _D), lambda i: (0, 0))],
        out_specs=pl.BlockSpec((1, 2, _BN), lambda i: (i, 0, 0)),
        out_shape=jax.ShapeDtypeStruct((_NB, 2, _BN), jnp.float32),
    )(x, w_rows)


def _edge_body(y_hbm, ei_hbm, out_hbm, ytab, srcv, dstv, acc,
               sem_y, sem_s, sem_d):
    wid = lax.axis_index("s") * 2 + lax.axis_index("c")
    base = wid * _EPW
    cp_y = [pltpu.async_copy(y_hbm.at[j, 0], ytab.at[pl.ds(j * _BN, _BN)],
                             sem_y) for j in range(_NB)]
    cp_s = pltpu.async_copy(ei_hbm.at[0, pl.ds(base, _EPW)], srcv, sem_s)
    cp_d = pltpu.async_copy(ei_hbm.at[1, pl.ds(base, _EPW)], dstv, sem_d)

    zero = jnp.zeros((_L,), jnp.float32)

    @plsc.parallel_loop(0, _N // _L, unroll=8)
    def _zero(i):
        acc[pl.ds(i * _L, _L)] = zero

    for cp in cp_y:
        cp.wait()
    cp_s.wait()
    cp_d.wait()

    @plsc.parallel_loop(0, _EPW // _L, unroll=16)
    def _edge_iter(i):
        s = srcv[pl.ds(i * _L, _L)]
        d = dstv[pl.ds(i * _L, _L)]
        v = plsc.load_gather(ytab, [s])
        plsc.addupdate_scatter(acc, [d], v)
    pltpu.sync_copy(acc, out_hbm.at[wid])


def _edge(y_flat, edge_index):
    mesh = plsc.VectorSubcoreMesh(core_axis_name="c", subcore_axis_name="s")
    f = pl.kernel(
        _edge_body,
        mesh=mesh,
        compiler_params=pltpu.CompilerParams(needs_layout_passes=False,
                                             use_tc_tiling_on_sc=False),
        out_type=jax.ShapeDtypeStruct((_NW, _N), jnp.float32),
        scratch_types=[pltpu.VMEM((_N,), jnp.float32),
                       pltpu.VMEM((_EPW,), jnp.int32),
                       pltpu.VMEM((_EPW,), jnp.int32),
                       pltpu.VMEM((_N,), jnp.float32),
                       pltpu.SemaphoreType.DMA,
                       pltpu.SemaphoreType.DMA,
                       pltpu.SemaphoreType.DMA],
    )
    return f(y_flat, edge_index)


def _pool_body(x_ref, parts_ref, batch_ref, y2_ref, brel_ref, out_ref,
               m_ref, d_ref, g_ref):
    i = pl.program_id(0)

    @pl.when(i == 0)
    def _init():
        m_ref[...] = jnp.full((_B, 1), -jnp.inf, jnp.float32)
        d_ref[...] = jnp.zeros((_B, 1), jnp.float32)
        g_ref[...] = jnp.zeros((_B, _D), jnp.float32)

    x = x_ref[...]
    parts = parts_ref[...].reshape(_NW, _BN)
    e_row = jnp.sum(parts, axis=0, keepdims=True)
    yroot_row = y2_ref[0, 1, :].reshape(1, _BN)
    xc = e_row + yroot_row + brel_ref[...]

    b_row = batch_ref[...].reshape(1, _BN)
    gids = lax.broadcasted_iota(jnp.int32, (_B, _BN), 0)
    P = b_row == gids
    Pf = P.astype(jnp.float32)

    m_old = m_ref[...]
    blk_max = jnp.max(jnp.where(P, xc, -jnp.inf), axis=1, keepdims=True)
    m_new = jnp.maximum(m_old, blk_max)
    scale = jnp.exp(jnp.where(m_new == -jnp.inf, 0.0, m_old - m_new))
    m_safe = jnp.where(m_new == -jnp.inf, 0.0, m_new)
    mrow = lax.dot_general(
        m_safe, Pf, (((0,), (0,)), ((), ())),
        precision=lax.Precision.HIGHEST,
        preferred_element_type=jnp.float32)
    ex_row = jnp.exp(xc - mrow)
    EX = Pf * ex_row
    d_ref[...] = d_ref[...] * scale + jnp.sum(EX, axis=1, keepdims=True)
    g_ref[...] = g_ref[...] * scale + jnp.dot(
        EX, x, precision=lax.Precision.HIGHEST,
        preferred_element_type=jnp.float32)
    m_ref[...] = m_new

    @pl.when(i == _NB - 1)
    def _fin():
        out_ref[...] = g_ref[...] / (d_ref[...] + 1e-16)


def _pool(x, parts, batch3, y2, brel):
    return pl.pallas_call(
        _pool_body,
        grid=(_NB,),
        in_specs=[pl.BlockSpec((_BN, _D), lambda i: (i, 0)),
                  pl.BlockSpec((_NW, 1, 1, _BN), lambda i: (0, i, 0, 0)),
                  pl.BlockSpec((1, 1, _BN), lambda i: (i, 0, 0)),
                  pl.BlockSpec((1, 2, _BN), lambda i: (i, 0, 0)),
                  pl.BlockSpec((1, 1), lambda i: (0, 0))],
        out_specs=pl.BlockSpec((_B, _D), lambda i: (0, 0)),
        out_shape=jax.ShapeDtypeStruct((_B, _D), jnp.float32),
        scratch_shapes=[pltpu.VMEM((_B, 1), jnp.float32),
                        pltpu.VMEM((_B, 1), jnp.float32),
                        pltpu.VMEM((_B, _D), jnp.float32)],
    )(x, parts, batch3, y2, brel)


def kernel(x, edge_index, batch, W_rel, b_rel, W_root):
    w_rows = jnp.concatenate([W_rel.reshape(1, _D), W_root.reshape(1, _D)],
                             axis=0)
    y2 = _proj(x, w_rows)
    parts = _edge(y2, edge_index)
    parts = parts.reshape(_NW, _NB, 1, _BN)
    batch3 = batch.reshape(_NB, 1, _BN)
    gx = _pool(x, parts, batch3, y2, b_rel.reshape(1, 1))
    return gx


# speedup vs baseline: 1.4679x; 1.0494x over previous
import functools

import jax
import jax.numpy as jnp
from jax import lax
from jax.experimental import pallas as pl
from jax.experimental.pallas import tpu as pltpu
from jax.experimental.pallas import tpu_sc as plsc

_N = 10000
_E = 320000
_D = 128
_B = 64
_BN = 5000
_NB = _N // _BN
_NW = 16
_EPW = _E // _NW
_L = 16


def _proj_body(x_ref, w_ref, y_ref):
    y_ref[...] = lax.dot_general(
        w_ref[...], x_ref[...], (((1,), (1,)), ((), ())),
        precision=lax.Precision.HIGHEST,
        preferred_element_type=jnp.float32).reshape(1, 2, _BN)


def _proj(x, w_rows):
    return pl.pallas_call(
        _proj_body,
        grid=(_NB,),
        in_specs=[pl.BlockSpec((_BN, _D), lambda i: (i, 0)),
                  pl.BlockSpec((2, _D), lambda i: (0, 0))],
        out_specs=pl.BlockSpec((1, 2, _BN), lambda i: (i, 0, 0)),
        out_shape=jax.ShapeDtypeStruct((_NB, 2, _BN), jnp.float32),
    )(x, w_rows)


def _edge_body(y_hbm, ei_hbm, out_hbm, ytab, srcv, dstv, acc,
               sem_y, sem_s, sem_d):
    wid = lax.axis_index("s") + lax.axis_index("c") * 16
    base = wid * _EPW
    cp_y = [pltpu.async_copy(y_hbm.at[j, 0], ytab.at[pl.ds(j * _BN, _BN)],
                             sem_y) for j in range(_NB)]
    cp_s = pltpu.async_copy(ei_hbm.at[0, pl.ds(base, _EPW)], srcv, sem_s)
    cp_d = pltpu.async_copy(ei_hbm.at[1, pl.ds(base, _EPW)], dstv, sem_d)

    zero = jnp.zeros((_L,), jnp.float32)

    @plsc.parallel_loop(0, _N // _L, unroll=8)
    def _zero(i):
        acc[pl.ds(i * _L, _L)] = zero

    for cp in cp_y:
        cp.wait()
    cp_s.wait()
    cp_d.wait()

    @plsc.parallel_loop(0, _EPW // _L, unroll=16)
    def _edge_iter(i):
        s = srcv[pl.ds(i * _L, _L)]
        d = dstv[pl.ds(i * _L, _L)]
        v = plsc.load_gather(ytab, [s])
        plsc.addupdate_scatter(acc, [d], v)
    pltpu.sync_copy(acc, out_hbm.at[wid])


def _edge(y_flat, edge_index):
    mesh = plsc.VectorSubcoreMesh(core_axis_name="c", subcore_axis_name="s", num_cores=1)
    f = pl.kernel(
        _edge_body,
        mesh=mesh,
        compiler_params=pltpu.CompilerParams(needs_layout_passes=False,
                                             use_tc_tiling_on_sc=False),
        out_type=jax.ShapeDtypeStruct((_NW, _N), jnp.float32),
        scratch_types=[pltpu.VMEM((_N,), jnp.float32),
                       pltpu.VMEM((_EPW,), jnp.int32),
                       pltpu.VMEM((_EPW,), jnp.int32),
                       pltpu.VMEM((_N,), jnp.float32),
                       pltpu.SemaphoreType.DMA,
                       pltpu.SemaphoreType.DMA,
                       pltpu.SemaphoreType.DMA],
    )
    return f(y_flat, edge_index)


def _pool_body(x_ref, parts_ref, batch_ref, y2_ref, brel_ref, out_ref,
               m_ref, d_ref, g_ref):
    i = pl.program_id(0)

    @pl.when(i == 0)
    def _init():
        m_ref[...] = jnp.full((_B, 1), -jnp.inf, jnp.float32)
        d_ref[...] = jnp.zeros((_B, 1), jnp.float32)
        g_ref[...] = jnp.zeros((_B, _D), jnp.float32)

    x = x_ref[...]
    parts = parts_ref[...].reshape(_NW, _BN)
    e_row = jnp.sum(parts, axis=0, keepdims=True)
    yroot_row = y2_ref[0, 1, :].reshape(1, _BN)
    xc = e_row + yroot_row + brel_ref[...]

    b_row = batch_ref[...].reshape(1, _BN)
    gids = lax.broadcasted_iota(jnp.int32, (_B, _BN), 0)
    P = b_row == gids
    Pf = P.astype(jnp.float32)

    m_old = m_ref[...]
    blk_max = jnp.max(jnp.where(P, xc, -jnp.inf), axis=1, keepdims=True)
    m_new = jnp.maximum(m_old, blk_max)
    scale = jnp.exp(jnp.where(m_new == -jnp.inf, 0.0, m_old - m_new))
    m_safe = jnp.where(m_new == -jnp.inf, 0.0, m_new)
    mrow = lax.dot_general(
        m_safe, Pf, (((0,), (0,)), ((), ())),
        precision=lax.Precision.HIGHEST,
        preferred_element_type=jnp.float32)
    ex_row = jnp.exp(xc - mrow)
    EX = Pf * ex_row
    d_ref[...] = d_ref[...] * scale + jnp.sum(EX, axis=1, keepdims=True)
    g_ref[...] = g_ref[...] * scale + jnp.dot(
        EX, x, precision=lax.Precision.HIGHEST,
        preferred_element_type=jnp.float32)
    m_ref[...] = m_new

    @pl.when(i == _NB - 1)
    def _fin():
        out_ref[...] = g_ref[...] / (d_ref[...] + 1e-16)


def _pool(x, parts, batch3, y2, brel):
    return pl.pallas_call(
        _pool_body,
        grid=(_NB,),
        in_specs=[pl.BlockSpec((_BN, _D), lambda i: (i, 0)),
                  pl.BlockSpec((_NW, 1, 1, _BN), lambda i: (0, i, 0, 0)),
                  pl.BlockSpec((1, 1, _BN), lambda i: (i, 0, 0)),
                  pl.BlockSpec((1, 2, _BN), lambda i: (i, 0, 0)),
                  pl.BlockSpec((1, 1), lambda i: (0, 0))],
        out_specs=pl.BlockSpec((_B, _D), lambda i: (0, 0)),
        out_shape=jax.ShapeDtypeStruct((_B, _D), jnp.float32),
        scratch_shapes=[pltpu.VMEM((_B, 1), jnp.float32),
                        pltpu.VMEM((_B, 1), jnp.float32),
                        pltpu.VMEM((_B, _D), jnp.float32)],
    )(x, parts, batch3, y2, brel)


def kernel(x, edge_index, batch, W_rel, b_rel, W_root):
    w_rows = jnp.concatenate([W_rel.reshape(1, _D), W_root.reshape(1, _D)],
                             axis=0)
    y2 = _proj(x, w_rows)
    parts = _edge(y2, edge_index)
    parts = parts.reshape(_NW, _NB, 1, _BN)
    batch3 = batch.reshape(_NB, 1, _BN)
    gx = _pool(x, parts, batch3, y2, b_rel.reshape(1, 1))
    return gx


# speedup vs baseline: 1.6786x; 1.1435x over previous
import functools

import jax
import jax.numpy as jnp
from jax import lax
from jax.experimental import pallas as pl
from jax.experimental.pallas import tpu as pltpu
from jax.experimental.pallas import tpu_sc as plsc

_N = 10000
_E = 320000
_D = 128
_B = 64
_BN = 5000
_NB = _N // _BN
_NW = 16
_EPW = _E // _NW
_L = 16


def _proj_body(x_ref, w_ref, y_ref):
    y_ref[...] = lax.dot_general(
        w_ref[...], x_ref[...], (((1,), (1,)), ((), ())),
        preferred_element_type=jnp.float32).reshape(1, 2, _BN)


def _proj(x, w_rows):
    return pl.pallas_call(
        _proj_body,
        grid=(_NB,),
        in_specs=[pl.BlockSpec((_BN, _D), lambda i: (i, 0)),
                  pl.BlockSpec((2, _D), lambda i: (0, 0))],
        out_specs=pl.BlockSpec((1, 2, _BN), lambda i: (i, 0, 0)),
        out_shape=jax.ShapeDtypeStruct((_NB, 2, _BN), jnp.float32),
    )(x, w_rows)


def _edge_body(y_hbm, ei_hbm, out_hbm, ytab, srcv, dstv, acc,
               sem_y, sem_s, sem_d):
    wid = lax.axis_index("s") + lax.axis_index("c") * 16
    base = wid * _EPW
    cp_y = [pltpu.async_copy(y_hbm.at[j, 0], ytab.at[pl.ds(j * _BN, _BN)],
                             sem_y) for j in range(_NB)]
    cp_s = pltpu.async_copy(ei_hbm.at[0, pl.ds(base, _EPW)], srcv, sem_s)
    cp_d = pltpu.async_copy(ei_hbm.at[1, pl.ds(base, _EPW)], dstv, sem_d)

    zero = jnp.zeros((_L,), jnp.float32)

    @plsc.parallel_loop(0, _N // _L, unroll=8)
    def _zero(i):
        acc[pl.ds(i * _L, _L)] = zero

    for cp in cp_y:
        cp.wait()
    cp_s.wait()
    cp_d.wait()

    @plsc.parallel_loop(0, _EPW // _L, unroll=16)
    def _edge_iter(i):
        s = srcv[pl.ds(i * _L, _L)]
        d = dstv[pl.ds(i * _L, _L)]
        v = plsc.load_gather(ytab, [s])
        plsc.addupdate_scatter(acc, [d], v)
    pltpu.sync_copy(acc, out_hbm.at[wid])


def _edge(y_flat, edge_index):
    mesh = plsc.VectorSubcoreMesh(core_axis_name="c", subcore_axis_name="s", num_cores=1)
    f = pl.kernel(
        _edge_body,
        mesh=mesh,
        compiler_params=pltpu.CompilerParams(needs_layout_passes=False,
                                             use_tc_tiling_on_sc=False),
        out_type=jax.ShapeDtypeStruct((_NW, _N), jnp.float32),
        scratch_types=[pltpu.VMEM((_N,), jnp.float32),
                       pltpu.VMEM((_EPW,), jnp.int32),
                       pltpu.VMEM((_EPW,), jnp.int32),
                       pltpu.VMEM((_N,), jnp.float32),
                       pltpu.SemaphoreType.DMA,
                       pltpu.SemaphoreType.DMA,
                       pltpu.SemaphoreType.DMA],
    )
    return f(y_flat, edge_index)


def _pool_body(x_ref, parts_ref, batch_ref, y2_ref, brel_ref, out_ref,
               m_ref, d_ref, g_ref):
    i = pl.program_id(0)

    @pl.when(i == 0)
    def _init():
        m_ref[...] = jnp.full((_B, 1), -jnp.inf, jnp.float32)
        d_ref[...] = jnp.zeros((_B, 1), jnp.float32)
        g_ref[...] = jnp.zeros((_B, _D), jnp.float32)

    x = x_ref[...]
    parts = parts_ref[...].reshape(_NW, _BN)
    e_row = jnp.sum(parts, axis=0, keepdims=True)
    yroot_row = y2_ref[0, 1, :].reshape(1, _BN)
    xc = e_row + yroot_row + brel_ref[...]

    b_row = batch_ref[...].reshape(1, _BN)
    gids = lax.broadcasted_iota(jnp.int32, (_B, _BN), 0)
    P = b_row == gids
    Pf = P.astype(jnp.float32)

    m_old = m_ref[...]
    blk_max = jnp.max(jnp.where(P, xc, -jnp.inf), axis=1, keepdims=True)
    m_new = jnp.maximum(m_old, blk_max)
    scale = jnp.exp(jnp.where(m_new == -jnp.inf, 0.0, m_old - m_new))
    m_safe = jnp.where(m_new == -jnp.inf, 0.0, m_new)
    mrow = lax.dot_general(
        m_safe, Pf, (((0,), (0,)), ((), ())),
        precision=lax.Precision.HIGHEST,
        preferred_element_type=jnp.float32)
    ex_row = jnp.exp(xc - mrow)
    EX = Pf * ex_row
    d_ref[...] = d_ref[...] * scale + jnp.sum(EX, axis=1, keepdims=True)
    g_ref[...] = g_ref[...] * scale + jnp.dot(
        EX, x, preferred_element_type=jnp.float32)
    m_ref[...] = m_new

    @pl.when(i == _NB - 1)
    def _fin():
        out_ref[...] = g_ref[...] / (d_ref[...] + 1e-16)


def _pool(x, parts, batch3, y2, brel):
    return pl.pallas_call(
        _pool_body,
        grid=(_NB,),
        in_specs=[pl.BlockSpec((_BN, _D), lambda i: (i, 0)),
                  pl.BlockSpec((_NW, 1, 1, _BN), lambda i: (0, i, 0, 0)),
                  pl.BlockSpec((1, 1, _BN), lambda i: (i, 0, 0)),
                  pl.BlockSpec((1, 2, _BN), lambda i: (i, 0, 0)),
                  pl.BlockSpec((1, 1), lambda i: (0, 0))],
        out_specs=pl.BlockSpec((_B, _D), lambda i: (0, 0)),
        out_shape=jax.ShapeDtypeStruct((_B, _D), jnp.float32),
        scratch_shapes=[pltpu.VMEM((_B, 1), jnp.float32),
                        pltpu.VMEM((_B, 1), jnp.float32),
                        pltpu.VMEM((_B, _D), jnp.float32)],
    )(x, parts, batch3, y2, brel)


def kernel(x, edge_index, batch, W_rel, b_rel, W_root):
    w_rows = jnp.concatenate([W_rel.reshape(1, _D), W_root.reshape(1, _D)],
                             axis=0)
    y2 = _proj(x, w_rows)
    parts = _edge(y2, edge_index)
    parts = parts.reshape(_NW, _NB, 1, _BN)
    batch3 = batch.reshape(_NB, 1, _BN)
    gx = _pool(x, parts, batch3, y2, b_rel.reshape(1, 1))
    return gx
